# baseline (speedup 1.0000x reference)
import functools

import numpy as np
import jax
import jax.numpy as jnp
from jax import lax
from jax.experimental import pallas as pl
from jax.experimental.pallas import tpu as pltpu
from jax.experimental.pallas import tpu_sc as plsc

_VOCAB = 1000000
_EMBED = 32
_HALF = 16
_B = 4
_S = 4096
_NC = 2
_NS = 16
_NW = _NC * _NS
_ROWS = _B * _S
_RPW = _ROWS // _NW
_CH = 4
_CHROWS = _RPW // _CH
_SPW = _RPW // _B

_theta32 = (1.0 / (10000.0 ** (np.arange(_HALF, dtype=np.float32) / np.float32(_HALF)))).astype(np.float32)
_ang32 = (np.arange(_S, dtype=np.float32)[:, None] * _theta32[None, :]).astype(np.float32)
_COS_TABLE = np.cos(_ang32.astype(np.float64)).astype(np.float32).reshape(-1)
_SIN_TABLE = np.sin(_ang32.astype(np.float64)).astype(np.float32).reshape(-1)

_mesh = plsc.VectorSubcoreMesh(core_axis_name="c", subcore_axis_name="s")


@functools.partial(
    pl.kernel,
    mesh=_mesh,
    compiler_params=pltpu.CompilerParams(use_tc_tiling_on_sc=False),
    out_type=jax.ShapeDtypeStruct((_ROWS, _EMBED), jnp.float32),
    scratch_types=[
        pltpu.VMEM((_CH, _CHROWS), jnp.int32),
        pltpu.VMEM((_RPW, _EMBED), jnp.float32),
        pltpu.VMEM((_SPW * _HALF,), jnp.float32),
        pltpu.VMEM((_SPW * _HALF,), jnp.float32),
        pltpu.SemaphoreType.DMA,
    ],
)
def _embed_rotary(idx_hbm, table_hbm, cos_hbm, sin_hbm, out_hbm,
                  idx_v, rows_v, cos_v, sin_v, sem):
    wid = lax.axis_index("s") * _NC + lax.axis_index("c")
    base = wid * _RPW
    pbase = wid * _SPW * _HALF

    pltpu.sync_copy(idx_hbm.at[pl.ds(wid * _CH, _CH)], idx_v)
    pltpu.sync_copy(cos_hbm.at[pl.ds(pbase, _SPW * _HALF)], cos_v)
    pltpu.sync_copy(sin_hbm.at[pl.ds(pbase, _SPW * _HALF)], sin_v)

    copies = []
    for k in range(_CH):
        copies.append(
            pltpu.async_copy(
                table_hbm.at[idx_v.at[k]],
                rows_v.at[pl.ds(k * _CHROWS, _CHROWS)],
                sem,
            )
        )
    for c in copies:
        c.wait()

    def body(j, carry):
        cos = cos_v[pl.ds(j * _HALF, _HALF)]
        sin = sin_v[pl.ds(j * _HALF, _HALF)]
        for b in range(_B):
            i = j * _B + b
            x1 = rows_v[i, 0:_HALF]
            x2 = rows_v[i, _HALF:_EMBED]
            rows_v[i, 0:_HALF] = x1 * cos - x2 * sin
            rows_v[i, _HALF:_EMBED] = x1 * sin + x2 * cos
        return carry

    lax.fori_loop(0, _SPW, body, 0)

    pltpu.sync_copy(rows_v, out_hbm.at[pl.ds(base, _RPW)])


def kernel(x, token_embedding):
    idx = x.T.reshape(_NW * _CH, _CHROWS)
    out = _embed_rotary(idx, token_embedding, jnp.asarray(_COS_TABLE),
                        jnp.asarray(_SIN_TABLE))
    return out.reshape(_S, _B, _EMBED)
